# baseline (speedup 1.0000x reference)
import functools

import jax
import jax.numpy as jnp
from jax import lax
from jax.experimental import pallas as pl
from jax.experimental.pallas import tpu as pltpu
from jax.experimental.pallas import tpu_sc as plsc

N = 100000
E = 1600000
DV = 128
DE = 16
B = 64
L = 16

NC = 2
NS = 16
NW = NC * NS

EPC = 512
NCH_E = E // EPC
TPT_E = 98

NPC = 160
NCH_N = N // NPC
TPT_N = 20

EB_HALF = E * 8
EBP = 520


def _sc_agg(v_flat, e_flat, src2d, batch):
    mesh = plsc.VectorSubcoreMesh(core_axis_name="c", subcore_axis_name="s")
    f32 = jnp.float32
    i32 = jnp.int32
    out_type = [
        jax.ShapeDtypeStruct((NW, B * L), f32),
        jax.ShapeDtypeStruct((NW, B), f32),
        jax.ShapeDtypeStruct((NW, B * L), f32),
        jax.ShapeDtypeStruct((NW, B * L), f32),
        jax.ShapeDtypeStruct((NW, B * DV), f32),
        jax.ShapeDtypeStruct((NW, B), f32),
        jax.ShapeDtypeStruct((NW, B * DV), f32),
        jax.ShapeDtypeStruct((NW, B * DV), f32),
    ]
    scratch = dict(
        src_b0=pltpu.VMEM((4, 128), i32),
        src_b1=pltpu.VMEM((4, 128), i32),
        seg_b0=pltpu.VMEM((EPC,), i32),
        seg_b1=pltpu.VMEM((EPC,), i32),
        e_b0=pltpu.VMEM((DE * EBP,), f32),
        e_b1=pltpu.VMEM((DE * EBP,), f32),
        nseg_b0=pltpu.VMEM((NPC,), i32),
        nseg_b1=pltpu.VMEM((NPC,), i32),
        v_b0=pltpu.VMEM((NPC * DV,), f32),
        v_b1=pltpu.VMEM((NPC * DV,), f32),
        a_es=pltpu.VMEM((B * L,), f32),
        a_ec=pltpu.VMEM((B,), f32),
        a_emin0=pltpu.VMEM((B * L,), f32),
        a_emin1=pltpu.VMEM((B * L,), f32),
        a_emax0=pltpu.VMEM((B * L,), f32),
        a_emax1=pltpu.VMEM((B * L,), f32),
        a_vs=pltpu.VMEM((B * DV,), f32),
        a_vc=pltpu.VMEM((B,), f32),
        a_vmin0=pltpu.VMEM((B * DV,), f32),
        a_vmin1=pltpu.VMEM((B * DV,), f32),
        a_vmax0=pltpu.VMEM((B * DV,), f32),
        a_vmax1=pltpu.VMEM((B * DV,), f32),
        sem_s0=pltpu.SemaphoreType.DMA,
        sem_s1=pltpu.SemaphoreType.DMA,
        sem_e0=pltpu.SemaphoreType.DMA,
        sem_e1=pltpu.SemaphoreType.DMA,
        sem_g0=pltpu.SemaphoreType.DMA,
        sem_g1=pltpu.SemaphoreType.DMA,
        sem_n0=pltpu.SemaphoreType.DMA,
        sem_n1=pltpu.SemaphoreType.DMA,
        sem_v0=pltpu.SemaphoreType.DMA,
        sem_v1=pltpu.SemaphoreType.DMA,
    )

    @functools.partial(
        pl.kernel, out_type=out_type, mesh=mesh, scratch_types=scratch,
        compiler_params=pltpu.CompilerParams(needs_layout_passes=False))
    def body(v_h, e_h, src_h, batch_h,
             e_sum_o, e_cnt_o, e_min_o, e_max_o,
             v_sum_o, v_cnt_o, v_min_o, v_max_o,
             src_b0, src_b1, seg_b0, seg_b1, e_b0, e_b1,
             nseg_b0, nseg_b1, v_b0, v_b1,
             a_es, a_ec, a_emin0, a_emin1, a_emax0, a_emax1,
             a_vs, a_vc, a_vmin0, a_vmin1, a_vmax0, a_vmax1,
             sem_s0, sem_s1, sem_e0, sem_e1, sem_g0, sem_g1,
             sem_n0, sem_n1, sem_v0, sem_v1):
        w = lax.axis_index("s") * NC + lax.axis_index("c")
        iota = lax.iota(i32, L)
        ones = jnp.ones((L,), f32)
        inf = jnp.full((L,), jnp.inf, f32)
        ninf = jnp.full((L,), -jnp.inf, f32)
        zeros = jnp.zeros((L,), f32)

        src_b = [src_b0, src_b1]
        seg_b = [seg_b0, seg_b1]
        e_b = [e_b0, e_b1]
        nseg_b = [nseg_b0, nseg_b1]
        v_b = [v_b0, v_b1]
        a_emin = [a_emin0, a_emin1]
        a_emax = [a_emax0, a_emax1]
        a_vmin = [a_vmin0, a_vmin1]
        a_vmax = [a_vmax0, a_vmax1]
        sem_s = [sem_s0, sem_s1]
        sem_e = [sem_e0, sem_e1]
        sem_g = [sem_g0, sem_g1]
        sem_n = [sem_n0, sem_n1]
        sem_v = [sem_v0, sem_v1]

        def initb(i, c):
            sl = pl.ds(i * L, L)
            a_es[sl] = zeros
            a_emin0[sl] = inf
            a_emin1[sl] = inf
            a_emax0[sl] = ninf
            a_emax1[sl] = ninf
            return c
        lax.fori_loop(0, B, initb, 0)

        def initc(i, c):
            sl = pl.ds(i * L, L)
            a_ec[sl] = zeros
            a_vc[sl] = zeros
            return c
        lax.fori_loop(0, B // L, initc, 0)

        def initv(i, c):
            sl = pl.ds(i * L, L)
            a_vs[sl] = zeros
            a_vmin0[sl] = inf
            a_vmin1[sl] = inf
            a_vmax0[sl] = ninf
            a_vmax1[sl] = ninf
            return c
        lax.fori_loop(0, B * DV // L, initv, 0)

        def e_issue_lin(k, p):
            pltpu.async_copy(src_h.at[pl.ds(k * 4, 4), :], src_b[p],
                             sem_s[p])
            for f in range(DE):
                fh, r = f // 8, f % 8
                for c in range(4):
                    pltpu.async_copy(
                        e_h.at[pl.ds(fh * EB_HALF + (k * 4 + c) * 1024
                                     + r * 128, 128)],
                        e_b[p].at[pl.ds(f * EBP + c * 128, 128)],
                        sem_e[p])

        def e_wait_lin_src(k, p):
            pltpu.make_async_copy(src_h.at[pl.ds(k * 4, 4), :], src_b[p],
                                  sem_s[p]).wait()

        def e_wait_lin_e(k, p):
            for f in range(DE):
                fh, r = f // 8, f % 8
                for c in range(4):
                    pltpu.make_async_copy(
                        e_h.at[pl.ds(fh * EB_HALF + (k * 4 + c) * 1024
                                     + r * 128, 128)],
                        e_b[p].at[pl.ds(f * EBP + c * 128, 128)],
                        sem_e[p]).wait()

        def e_issue_gather(p):
            for r in range(4):
                pltpu.async_copy(batch_h.at[src_b[p].at[r]],
                                 seg_b[p].at[pl.ds(r * 128, 128)], sem_g[p])

        def e_wait_gather(p):
            for r in range(4):
                pltpu.make_async_copy(batch_h.at[src_b[p].at[r]],
                                      seg_b[p].at[pl.ds(r * 128, 128)],
                                      sem_g[p]).wait()

        fvec = iota * EBP

        def e_compute(p):
            for je in range(4):
                def egrp(gj, cc, _je=je):
                    l0 = gj * 16
                    segv = seg_b[p][pl.ds(_je * 128 + l0, 16)]
                    segv16 = segv * L
                    plsc.addupdate_scatter(a_ec, [segv], ones)
                    for j in range(L):
                        idx = segv16[j] + iota
                        xe = plsc.load_gather(
                            e_b[p], [fvec + (l0 + (_je * 128 + j))])
                        plsc.addupdate_scatter(a_es, [idx], xe)
                        am = a_emin[j % 2]
                        mn = plsc.load_gather(am, [idx])
                        plsc.store_scatter(am, [idx], jnp.minimum(mn, xe))
                        ax = a_emax[j % 2]
                        mx = plsc.load_gather(ax, [idx])
                        plsc.store_scatter(ax, [idx], jnp.maximum(mx, xe))
                    return cc
                lax.fori_loop(0, 8, egrp, 0)

        e_issue_lin(w, 0)
        e_issue_lin(w + NW, 1)
        e_wait_lin_src(w, 0)
        e_issue_gather(0)

        def etrip(u, c):
            for p in range(2):
                cc = 2 * u + p
                k = w + NW * cc
                k1 = k + NW
                k2 = k + 2 * NW

                @pl.when(k < NCH_E)
                def _():
                    e_wait_gather(p)
                    e_wait_lin_e(k, p)

                @pl.when(k1 < NCH_E)
                def _():
                    e_wait_lin_src(k1, 1 - p)
                    e_issue_gather(1 - p)

                @pl.when(k < NCH_E)
                def _():
                    e_compute(p)

                @pl.when(k2 < NCH_E)
                def _():
                    e_issue_lin(k2, p)
            return c
        lax.fori_loop(0, TPT_E // 2, etrip, 0)

        def n_issue(k, p):
            pltpu.async_copy(batch_h.at[pl.ds(k * NPC, NPC)], nseg_b[p],
                             sem_n[p])
            pltpu.async_copy(v_h.at[pl.ds(k * NPC * DV, NPC * DV)], v_b[p],
                             sem_v[p])

        def n_wait(k, p):
            pltpu.make_async_copy(batch_h.at[pl.ds(k * NPC, NPC)],
                                  nseg_b[p], sem_n[p]).wait()
            pltpu.make_async_copy(v_h.at[pl.ds(k * NPC * DV, NPC * DV)],
                                  v_b[p], sem_v[p]).wait()

        fconst = [f * L + iota for f in range(DV // L)]

        def n_compute(p):
            def ngrp(gi, cc):
                nsegv = nseg_b[p][pl.ds(gi * 16, 16)]
                plsc.addupdate_scatter(a_vc, [nsegv], ones)
                n0 = gi * 16
                s0 = nsegv[0]
                s15 = nsegv[15]
                same = s0 == s15

                @pl.when(same)
                def _():
                    base = s0 * DV
                    for f in range(DV // L):
                        x = v_b[p][pl.ds(n0 * DV + f * L, L)]
                        sm, mn, mx = x, x, x
                        for j in range(1, L):
                            y = v_b[p][pl.ds((n0 + j) * DV + f * L, L)]
                            sm = sm + y
                            mn = jnp.minimum(mn, y)
                            mx = jnp.maximum(mx, y)
                        idf = base + fconst[f]
                        plsc.addupdate_scatter(a_vs, [idf], sm)
                        am = a_vmin[f % 2]
                        ax = a_vmax[f % 2]
                        cmn = plsc.load_gather(am, [idf])
                        plsc.store_scatter(am, [idf],
                                           jnp.minimum(cmn, mn))
                        cmx = plsc.load_gather(ax, [idf])
                        plsc.store_scatter(ax, [idf],
                                           jnp.maximum(cmx, mx))

                @pl.when(jnp.logical_not(same))
                def _():
                    for j in range(L):
                        n = n0 + j
                        s = nsegv[j]
                        base = s * DV
                        am = a_vmin[j % 2]
                        ax = a_vmax[j % 2]
                        for f in range(DV // L):
                            idf = base + fconst[f]
                            xv = v_b[p][pl.ds(n * DV + f * L, L)]
                            plsc.addupdate_scatter(a_vs, [idf], xv)
                            mn = plsc.load_gather(am, [idf])
                            plsc.store_scatter(am, [idf],
                                               jnp.minimum(mn, xv))
                            mx = plsc.load_gather(ax, [idf])
                            plsc.store_scatter(ax, [idf],
                                               jnp.maximum(mx, xv))
                return cc
            lax.fori_loop(0, NPC // 16, ngrp, 0)

        n_issue(w, 0)
        n_issue(w + NW, 1)

        def ntrip(u, c):
            for p in range(2):
                cc = 2 * u + p
                k = w + NW * cc
                k2 = k + 2 * NW

                @pl.when(k < NCH_N)
                def _():
                    n_wait(k, p)
                    n_compute(p)

                @pl.when(k2 < NCH_N)
                def _():
                    n_issue(k2, p)
            return c
        lax.fori_loop(0, TPT_N // 2, ntrip, 0)

        def mrg_e(i, c):
            sl = pl.ds(i * L, L)
            a_emin0[sl] = jnp.minimum(a_emin0[sl], a_emin1[sl])
            a_emax0[sl] = jnp.maximum(a_emax0[sl], a_emax1[sl])
            return c
        lax.fori_loop(0, B, mrg_e, 0)

        def mrg_v(i, c):
            sl = pl.ds(i * L, L)
            a_vmin0[sl] = jnp.minimum(a_vmin0[sl], a_vmin1[sl])
            a_vmax0[sl] = jnp.maximum(a_vmax0[sl], a_vmax1[sl])
            return c
        lax.fori_loop(0, B * DV // L, mrg_v, 0)

        pltpu.sync_copy(a_es, e_sum_o.at[w])
        pltpu.sync_copy(a_ec, e_cnt_o.at[w])
        pltpu.sync_copy(a_emin0, e_min_o.at[w])
        pltpu.sync_copy(a_emax0, e_max_o.at[w])
        pltpu.sync_copy(a_vs, v_sum_o.at[w])
        pltpu.sync_copy(a_vc, v_cnt_o.at[w])
        pltpu.sync_copy(a_vmin0, v_min_o.at[w])
        pltpu.sync_copy(a_vmax0, v_max_o.at[w])

    outs = body(v_flat, e_flat, src2d, batch)
    return outs


def _tc_finish(g, W1, b1, W2, b2, parts):
    f32 = jnp.float32
    (e_sum_p, e_cnt_p, e_min_p, e_max_p,
     v_sum_p, v_cnt_p, v_min_p, v_max_p) = parts

    def body(g_r, W1_r, b1_r, W2_r, b2_r,
             es_r, ec_r, emin_r, emax_r, vs_r, vc_r, vmin_r, vmax_r, y_r):
        ec = jnp.sum(ec_r[...], axis=0)
        cnt_e = ec[:, None]
        es = jnp.sum(es_r[...], axis=0)
        emn = jnp.min(emin_r[...], axis=0)
        emx = jnp.max(emax_r[...], axis=0)
        has_e = cnt_e > 0
        e_mean = jnp.where(has_e, es / jnp.maximum(cnt_e, 1.0), 0.0)
        emn = jnp.where(has_e, emn, 0.0)
        emx = jnp.where(has_e, emx, 0.0)

        vc = jnp.sum(vc_r[...], axis=0)
        cnt_v = vc[:, None]
        vs = jnp.sum(vs_r[...], axis=0)
        vmn = jnp.min(vmin_r[...], axis=0)
        vmx = jnp.max(vmax_r[...], axis=0)
        has_v = cnt_v > 0
        v_mean = jnp.where(has_v, vs / jnp.maximum(cnt_v, 1.0), 0.0)
        vmn = jnp.where(has_v, vmn, 0.0)
        vmx = jnp.where(has_v, vmx, 0.0)

        W1v = W1_r[...]

        def mm(x, lo, size):
            return jnp.dot(x, W1v[lo:lo + size, :],
                           preferred_element_type=f32)

        acc = mm(g_r[...], 0, 32)
        acc += mm(emn, 32, 16)
        acc += mm(e_mean, 48, 16)
        acc += mm(es, 64, 16)
        acc += mm(emx, 80, 16)
        acc += mm(vmn, 96, 128)
        acc += mm(v_mean, 224, 128)
        acc += mm(vs, 352, 128)
        acc += mm(vmx, 480, 128)
        h = jnp.maximum(acc + b1_r[...].reshape(1, -1), 0.0)
        y = jnp.dot(h, W2_r[...], preferred_element_type=f32)
        y_r[...] = y + b2_r[...].reshape(1, -1)

    return pl.pallas_call(
        body,
        out_shape=jax.ShapeDtypeStruct((B, 128), f32),
    )(g, W1, b1, W2, b2,
      e_sum_p, e_cnt_p, e_min_p, e_max_p,
      v_sum_p, v_cnt_p, v_min_p, v_max_p)


def kernel(v_attr, edgeij_pair, e_attr, g, batch, W1, b1, W2, b2):
    v_flat = v_attr.reshape(-1)
    e_flat = (e_attr.T.reshape(DE // 8, 8, E // 128, 128)
              .transpose(0, 2, 1, 3).reshape(-1))
    src2d = edgeij_pair[0].reshape(E // 128, 128)
    parts = _sc_agg(v_flat, e_flat, src2d, batch)
    parts = [
        parts[0].reshape(NW, B, L), parts[1].reshape(NW, B),
        parts[2].reshape(NW, B, L), parts[3].reshape(NW, B, L),
        parts[4].reshape(NW, B, DV), parts[5].reshape(NW, B),
        parts[6].reshape(NW, B, DV), parts[7].reshape(NW, B, DV),
    ]
    return _tc_finish(g, W1, b1, W2, b2, parts)


# speedup vs baseline: 1.0209x; 1.0209x over previous
import functools

import jax
import jax.numpy as jnp
from jax import lax
from jax.experimental import pallas as pl
from jax.experimental.pallas import tpu as pltpu
from jax.experimental.pallas import tpu_sc as plsc

N = 100000
E = 1600000
DV = 128
DE = 16
B = 64
L = 16

NC = 2
NS = 16
NW = NC * NS

EPC = 512
NCH_E = E // EPC
TPT_E = 98

NPC = 160
NCH_N = N // NPC
TPT_N = 20

EB_HALF = E * 8
SKP = 17


def _sc_agg(v_flat, e_flat, src2d, batch):
    mesh = plsc.VectorSubcoreMesh(core_axis_name="c", subcore_axis_name="s")
    f32 = jnp.float32
    i32 = jnp.int32
    out_type = [
        jax.ShapeDtypeStruct((NW, B * L), f32),
        jax.ShapeDtypeStruct((NW, B), f32),
        jax.ShapeDtypeStruct((NW, B * L), f32),
        jax.ShapeDtypeStruct((NW, B * L), f32),
        jax.ShapeDtypeStruct((NW, B * DV), f32),
        jax.ShapeDtypeStruct((NW, B), f32),
        jax.ShapeDtypeStruct((NW, B * DV), f32),
        jax.ShapeDtypeStruct((NW, B * DV), f32),
    ]
    scratch = dict(
        src_b0=pltpu.VMEM((4, 128), i32),
        src_b1=pltpu.VMEM((4, 128), i32),
        seg_b0=pltpu.VMEM((EPC,), i32),
        seg_b1=pltpu.VMEM((EPC,), i32),
        e_b0=pltpu.VMEM((EPC * DE,), f32),
        e_b1=pltpu.VMEM((EPC * DE,), f32),
        nseg_b0=pltpu.VMEM((NPC,), i32),
        nseg_b1=pltpu.VMEM((NPC,), i32),
        v_b0=pltpu.VMEM((NPC * DV,), f32),
        v_b1=pltpu.VMEM((NPC * DV,), f32),
        a_es=pltpu.VMEM((B * L,), f32),
        a_ec=pltpu.VMEM((B,), f32),
        sk=pltpu.VMEM((16 * SKP,), f32),
        a_emin0=pltpu.VMEM((B * L,), f32),
        a_emin1=pltpu.VMEM((B * L,), f32),
        a_emax0=pltpu.VMEM((B * L,), f32),
        a_emax1=pltpu.VMEM((B * L,), f32),
        a_vs=pltpu.VMEM((B * DV,), f32),
        a_vc=pltpu.VMEM((B,), f32),
        a_vmin0=pltpu.VMEM((B * DV,), f32),
        a_vmin1=pltpu.VMEM((B * DV,), f32),
        a_vmax0=pltpu.VMEM((B * DV,), f32),
        a_vmax1=pltpu.VMEM((B * DV,), f32),
        sem_s0=pltpu.SemaphoreType.DMA,
        sem_s1=pltpu.SemaphoreType.DMA,
        sem_e0=pltpu.SemaphoreType.DMA,
        sem_e1=pltpu.SemaphoreType.DMA,
        sem_g0=pltpu.SemaphoreType.DMA,
        sem_g1=pltpu.SemaphoreType.DMA,
        sem_n0=pltpu.SemaphoreType.DMA,
        sem_n1=pltpu.SemaphoreType.DMA,
        sem_v0=pltpu.SemaphoreType.DMA,
        sem_v1=pltpu.SemaphoreType.DMA,
    )

    @functools.partial(
        pl.kernel, out_type=out_type, mesh=mesh, scratch_types=scratch,
        compiler_params=pltpu.CompilerParams(needs_layout_passes=False))
    def body(v_h, e_h, src_h, batch_h,
             e_sum_o, e_cnt_o, e_min_o, e_max_o,
             v_sum_o, v_cnt_o, v_min_o, v_max_o,
             src_b0, src_b1, seg_b0, seg_b1, e_b0, e_b1,
             nseg_b0, nseg_b1, v_b0, v_b1,
             a_es, a_ec, sk, a_emin0, a_emin1, a_emax0, a_emax1,
             a_vs, a_vc, a_vmin0, a_vmin1, a_vmax0, a_vmax1,
             sem_s0, sem_s1, sem_e0, sem_e1, sem_g0, sem_g1,
             sem_n0, sem_n1, sem_v0, sem_v1):
        w = lax.axis_index("s") * NC + lax.axis_index("c")
        iota = lax.iota(i32, L)
        ones = jnp.ones((L,), f32)
        inf = jnp.full((L,), jnp.inf, f32)
        ninf = jnp.full((L,), -jnp.inf, f32)
        zeros = jnp.zeros((L,), f32)

        src_b = [src_b0, src_b1]
        seg_b = [seg_b0, seg_b1]
        e_b = [e_b0, e_b1]
        nseg_b = [nseg_b0, nseg_b1]
        v_b = [v_b0, v_b1]
        a_emin = [a_emin0, a_emin1]
        a_emax = [a_emax0, a_emax1]
        a_vmin = [a_vmin0, a_vmin1]
        a_vmax = [a_vmax0, a_vmax1]
        sem_s = [sem_s0, sem_s1]
        sem_e = [sem_e0, sem_e1]
        sem_g = [sem_g0, sem_g1]
        sem_n = [sem_n0, sem_n1]
        sem_v = [sem_v0, sem_v1]

        def initb(i, c):
            sl = pl.ds(i * L, L)
            a_es[sl] = zeros
            a_emin0[sl] = inf
            a_emin1[sl] = inf
            a_emax0[sl] = ninf
            a_emax1[sl] = ninf
            return c
        lax.fori_loop(0, B, initb, 0)

        def initc(i, c):
            sl = pl.ds(i * L, L)
            a_ec[sl] = zeros
            a_vc[sl] = zeros
            return c
        lax.fori_loop(0, B // L, initc, 0)

        def initv(i, c):
            sl = pl.ds(i * L, L)
            a_vs[sl] = zeros
            a_vmin0[sl] = inf
            a_vmin1[sl] = inf
            a_vmax0[sl] = ninf
            a_vmax1[sl] = ninf
            return c
        lax.fori_loop(0, B * DV // L, initv, 0)

        def e_issue_lin(k, p):
            pltpu.async_copy(src_h.at[pl.ds(k * 4, 4), :], src_b[p],
                             sem_s[p])
            for fh in range(2):
                pltpu.async_copy(
                    e_h.at[pl.ds(fh * EB_HALF + k * 4096, 4096)],
                    e_b[p].at[pl.ds(fh * 4096, 4096)], sem_e[p])

        def e_wait_lin_src(k, p):
            pltpu.make_async_copy(src_h.at[pl.ds(k * 4, 4), :], src_b[p],
                                  sem_s[p]).wait()

        def e_wait_lin_e(k, p):
            for fh in range(2):
                pltpu.make_async_copy(
                    e_h.at[pl.ds(fh * EB_HALF + k * 4096, 4096)],
                    e_b[p].at[pl.ds(fh * 4096, 4096)], sem_e[p]).wait()

        def e_issue_gather(p):
            for r in range(4):
                pltpu.async_copy(batch_h.at[src_b[p].at[r]],
                                 seg_b[p].at[pl.ds(r * 128, 128)], sem_g[p])

        def e_wait_gather(p):
            for r in range(4):
                pltpu.make_async_copy(batch_h.at[src_b[p].at[r]],
                                      seg_b[p].at[pl.ds(r * 128, 128)],
                                      sem_g[p]).wait()

        idx17 = iota * SKP

        def e_compute(p):
            for je in range(4):
                def egrp(gj, cc, _je=je):
                    l0 = gj * 16
                    segv = seg_b[p][pl.ds(_je * 128 + l0, 16)]
                    segv16 = segv * L
                    for f in range(DE):
                        fh, r = f // 8, f % 8
                        vf = e_b[p][pl.ds(fh * 4096 + _je * 1024
                                          + r * 128 + l0, L)]
                        plsc.store_scatter(sk, [idx17 + f], vf)
                    plsc.addupdate_scatter(a_ec, [segv], ones)
                    for j in range(L):
                        idx = segv16[j] + iota
                        xe = plsc.load_gather(sk, [iota + (j * SKP)])
                        plsc.addupdate_scatter(a_es, [idx], xe)
                        am = a_emin[j % 2]
                        mn = plsc.load_gather(am, [idx])
                        plsc.store_scatter(am, [idx], jnp.minimum(mn, xe))
                        ax = a_emax[j % 2]
                        mx = plsc.load_gather(ax, [idx])
                        plsc.store_scatter(ax, [idx], jnp.maximum(mx, xe))
                    return cc
                lax.fori_loop(0, 8, egrp, 0)

        e_issue_lin(w, 0)
        e_issue_lin(w + NW, 1)
        e_wait_lin_src(w, 0)
        e_issue_gather(0)

        def etrip(u, c):
            for p in range(2):
                cc = 2 * u + p
                k = w + NW * cc
                k1 = k + NW
                k2 = k + 2 * NW

                @pl.when(k < NCH_E)
                def _():
                    e_wait_gather(p)
                    e_wait_lin_e(k, p)

                @pl.when(k1 < NCH_E)
                def _():
                    e_wait_lin_src(k1, 1 - p)
                    e_issue_gather(1 - p)

                @pl.when(k < NCH_E)
                def _():
                    e_compute(p)

                @pl.when(k2 < NCH_E)
                def _():
                    e_issue_lin(k2, p)
            return c
        lax.fori_loop(0, TPT_E // 2, etrip, 0)

        def n_issue(k, p):
            pltpu.async_copy(batch_h.at[pl.ds(k * NPC, NPC)], nseg_b[p],
                             sem_n[p])
            pltpu.async_copy(v_h.at[pl.ds(k * NPC * DV, NPC * DV)], v_b[p],
                             sem_v[p])

        def n_wait(k, p):
            pltpu.make_async_copy(batch_h.at[pl.ds(k * NPC, NPC)],
                                  nseg_b[p], sem_n[p]).wait()
            pltpu.make_async_copy(v_h.at[pl.ds(k * NPC * DV, NPC * DV)],
                                  v_b[p], sem_v[p]).wait()

        fconst = [f * L + iota for f in range(DV // L)]

        def n_compute(p):
            def ngrp(gi, cc):
                nsegv = nseg_b[p][pl.ds(gi * 16, 16)]
                plsc.addupdate_scatter(a_vc, [nsegv], ones)
                n0 = gi * 16
                s0 = nsegv[0]
                s15 = nsegv[15]
                same = s0 == s15

                @pl.when(same)
                def _():
                    base = s0 * DV
                    for f in range(DV // L):
                        x = v_b[p][pl.ds(n0 * DV + f * L, L)]
                        sm, mn, mx = x, x, x
                        for j in range(1, L):
                            y = v_b[p][pl.ds((n0 + j) * DV + f * L, L)]
                            sm = sm + y
                            mn = jnp.minimum(mn, y)
                            mx = jnp.maximum(mx, y)
                        idf = base + fconst[f]
                        plsc.addupdate_scatter(a_vs, [idf], sm)
                        am = a_vmin[f % 2]
                        ax = a_vmax[f % 2]
                        cmn = plsc.load_gather(am, [idf])
                        plsc.store_scatter(am, [idf],
                                           jnp.minimum(cmn, mn))
                        cmx = plsc.load_gather(ax, [idf])
                        plsc.store_scatter(ax, [idf],
                                           jnp.maximum(cmx, mx))

                @pl.when(jnp.logical_not(same))
                def _():
                    for j in range(L):
                        n = n0 + j
                        s = nsegv[j]
                        base = s * DV
                        am = a_vmin[j % 2]
                        ax = a_vmax[j % 2]
                        for f in range(DV // L):
                            idf = base + fconst[f]
                            xv = v_b[p][pl.ds(n * DV + f * L, L)]
                            plsc.addupdate_scatter(a_vs, [idf], xv)
                            mn = plsc.load_gather(am, [idf])
                            plsc.store_scatter(am, [idf],
                                               jnp.minimum(mn, xv))
                            mx = plsc.load_gather(ax, [idf])
                            plsc.store_scatter(ax, [idf],
                                               jnp.maximum(mx, xv))
                return cc
            lax.fori_loop(0, NPC // 16, ngrp, 0)

        n_issue(w, 0)
        n_issue(w + NW, 1)

        def ntrip(u, c):
            for p in range(2):
                cc = 2 * u + p
                k = w + NW * cc
                k2 = k + 2 * NW

                @pl.when(k < NCH_N)
                def _():
                    n_wait(k, p)
                    n_compute(p)

                @pl.when(k2 < NCH_N)
                def _():
                    n_issue(k2, p)
            return c
        lax.fori_loop(0, TPT_N // 2, ntrip, 0)

        def mrg_e(i, c):
            sl = pl.ds(i * L, L)
            a_emin0[sl] = jnp.minimum(a_emin0[sl], a_emin1[sl])
            a_emax0[sl] = jnp.maximum(a_emax0[sl], a_emax1[sl])
            return c
        lax.fori_loop(0, B, mrg_e, 0)

        def mrg_v(i, c):
            sl = pl.ds(i * L, L)
            a_vmin0[sl] = jnp.minimum(a_vmin0[sl], a_vmin1[sl])
            a_vmax0[sl] = jnp.maximum(a_vmax0[sl], a_vmax1[sl])
            return c
        lax.fori_loop(0, B * DV // L, mrg_v, 0)

        pltpu.sync_copy(a_es, e_sum_o.at[w])
        pltpu.sync_copy(a_ec, e_cnt_o.at[w])
        pltpu.sync_copy(a_emin0, e_min_o.at[w])
        pltpu.sync_copy(a_emax0, e_max_o.at[w])
        pltpu.sync_copy(a_vs, v_sum_o.at[w])
        pltpu.sync_copy(a_vc, v_cnt_o.at[w])
        pltpu.sync_copy(a_vmin0, v_min_o.at[w])
        pltpu.sync_copy(a_vmax0, v_max_o.at[w])

    outs = body(v_flat, e_flat, src2d, batch)
    return outs


def _tc_finish(g, W1, b1, W2, b2, parts):
    f32 = jnp.float32
    (e_sum_p, e_cnt_p, e_min_p, e_max_p,
     v_sum_p, v_cnt_p, v_min_p, v_max_p) = parts

    def body(g_r, W1_r, b1_r, W2_r, b2_r,
             es_r, ec_r, emin_r, emax_r, vs_r, vc_r, vmin_r, vmax_r, y_r):
        ec = jnp.sum(ec_r[...], axis=0)
        cnt_e = ec[:, None]
        es = jnp.sum(es_r[...], axis=0)
        emn = jnp.min(emin_r[...], axis=0)
        emx = jnp.max(emax_r[...], axis=0)
        has_e = cnt_e > 0
        e_mean = jnp.where(has_e, es / jnp.maximum(cnt_e, 1.0), 0.0)
        emn = jnp.where(has_e, emn, 0.0)
        emx = jnp.where(has_e, emx, 0.0)

        vc = jnp.sum(vc_r[...], axis=0)
        cnt_v = vc[:, None]
        vs = jnp.sum(vs_r[...], axis=0)
        vmn = jnp.min(vmin_r[...], axis=0)
        vmx = jnp.max(vmax_r[...], axis=0)
        has_v = cnt_v > 0
        v_mean = jnp.where(has_v, vs / jnp.maximum(cnt_v, 1.0), 0.0)
        vmn = jnp.where(has_v, vmn, 0.0)
        vmx = jnp.where(has_v, vmx, 0.0)

        W1v = W1_r[...]

        def mm(x, lo, size):
            return jnp.dot(x, W1v[lo:lo + size, :],
                           preferred_element_type=f32)

        acc = mm(g_r[...], 0, 32)
        acc += mm(emn, 32, 16)
        acc += mm(e_mean, 48, 16)
        acc += mm(es, 64, 16)
        acc += mm(emx, 80, 16)
        acc += mm(vmn, 96, 128)
        acc += mm(v_mean, 224, 128)
        acc += mm(vs, 352, 128)
        acc += mm(vmx, 480, 128)
        h = jnp.maximum(acc + b1_r[...].reshape(1, -1), 0.0)
        y = jnp.dot(h, W2_r[...], preferred_element_type=f32)
        y_r[...] = y + b2_r[...].reshape(1, -1)

    return pl.pallas_call(
        body,
        out_shape=jax.ShapeDtypeStruct((B, 128), f32),
    )(g, W1, b1, W2, b2,
      e_sum_p, e_cnt_p, e_min_p, e_max_p,
      v_sum_p, v_cnt_p, v_min_p, v_max_p)


def kernel(v_attr, edgeij_pair, e_attr, g, batch, W1, b1, W2, b2):
    v_flat = v_attr.reshape(-1)
    e_flat = (e_attr.T.reshape(DE // 8, 8, E // 128, 128)
              .transpose(0, 2, 1, 3).reshape(-1))
    src2d = edgeij_pair[0].reshape(E // 128, 128)
    parts = _sc_agg(v_flat, e_flat, src2d, batch)
    parts = [
        parts[0].reshape(NW, B, L), parts[1].reshape(NW, B),
        parts[2].reshape(NW, B, L), parts[3].reshape(NW, B, L),
        parts[4].reshape(NW, B, DV), parts[5].reshape(NW, B),
        parts[6].reshape(NW, B, DV), parts[7].reshape(NW, B, DV),
    ]
    return _tc_finish(g, W1, b1, W2, b2, parts)
